# baseline (speedup 1.0000x reference)
import functools

import jax
import jax.numpy as jnp
from jax.experimental import pallas as pl
from jax.experimental.pallas import tpu as pltpu

_EPS = 1e-5
_VMEM_LIMIT = 64 * 1024 * 1024
_PAIRS = ((0, 0), (0, 1), (0, 2), (1, 1), (1, 2), (2, 2))


def _moments_kernel(x_ref, m9_ref, mom_ref, *, nb, cin, h_sub, n_steps):
    @pl.when(pl.program_id(0) == 0)
    def _init():
        mom_ref[...] = jnp.zeros_like(mom_ref)

    x = x_ref[...]
    for n in range(nb):
        for c in range(cin):
            mom_ref[c, :, :] += jnp.sum(x[n, c].reshape(h_sub, 8, -1), axis=0)
        for k, (a, b) in enumerate(_PAIRS):
            mom_ref[cin + k, :, :] += jnp.sum(
                (x[n, a] * x[n, b]).reshape(h_sub, 8, -1), axis=0)

    @pl.when(pl.program_id(0) == n_steps - 1)
    def _finalize():
        m9_ref[...] = jnp.sum(mom_ref[...], axis=(1, 2))


def _conv_kernel(x_ref, w_ref, shift_ref, o_ref, *, nb, cin, cout2):
    for n in range(nb):
        xs = [x_ref[n, c] for c in range(cin)]
        for o in range(cout2):
            acc = xs[0] * w_ref[o, 0] + shift_ref[o]
            for c in range(1, cin):
                acc = acc + xs[c] * w_ref[o, c]
            o_ref[n, o, :, :] = acc


def kernel(x_nchw, w1, b1, w2, b2, gamma, beta):
    del b1, b2
    N, Cin, H, W = x_nchw.shape
    Cout = w1.shape[0]
    Cout2 = 2 * Cout
    M = N * H * W

    x = x_nchw.astype(jnp.float32)
    w_cat = jnp.concatenate(
        [w1.reshape(Cout, Cin), w2.reshape(Cout, Cin)], axis=0
    ).astype(jnp.float32)

    NB = 4 if N % 4 == 0 else 1
    NB1 = 16 if N % 16 == 0 else NB
    n_planes = Cin + len(_PAIRS)
    m9 = pl.pallas_call(
        functools.partial(_moments_kernel, nb=NB1, cin=Cin, h_sub=H // 8,
                          n_steps=N // NB1),
        out_shape=jax.ShapeDtypeStruct((n_planes,), jnp.float32),
        grid=(N // NB1,),
        in_specs=[pl.BlockSpec((NB1, Cin, H, W),
                               lambda s: (s, 0, 0, 0))],
        out_specs=pl.BlockSpec((n_planes,), lambda s: (0,)),
        scratch_shapes=[pltpu.VMEM((n_planes, 8, W), jnp.float32)],
        compiler_params=pltpu.CompilerParams(
            dimension_semantics=("arbitrary",),
            vmem_limit_bytes=_VMEM_LIMIT),
    )(x)

    sum_x = m9[:Cin].reshape(Cin, 1)
    iu = jnp.array([[0, 1, 2], [1, 3, 4], [2, 4, 5]])
    sxx = m9[Cin:][iu]
    mean_x = sum_x / M
    cov_x = sxx / M - mean_x @ mean_x.T
    mean_y = w_cat @ mean_x
    var_y = jnp.maximum(
        jnp.sum((w_cat @ cov_x) * w_cat, axis=1, keepdims=True), 0.0)
    scale = gamma.astype(jnp.float32).reshape(Cout2, 1) * jax.lax.rsqrt(var_y + _EPS)
    shift = (beta.astype(jnp.float32).reshape(Cout2, 1) - mean_y * scale).reshape(Cout2)
    w_scaled = w_cat * scale

    out = pl.pallas_call(
        functools.partial(_conv_kernel, nb=NB, cin=Cin, cout2=Cout2),
        out_shape=jax.ShapeDtypeStruct((N, Cout2, H, W), jnp.float32),
        grid=(N // NB,),
        in_specs=[
            pl.BlockSpec((NB, Cin, H, W), lambda n: (n, 0, 0, 0)),
            pl.BlockSpec(memory_space=pltpu.SMEM),
            pl.BlockSpec(memory_space=pltpu.SMEM),
        ],
        out_specs=pl.BlockSpec((NB, Cout2, H, W), lambda n: (n, 0, 0, 0)),
        compiler_params=pltpu.CompilerParams(
            dimension_semantics=("parallel",),
            vmem_limit_bytes=_VMEM_LIMIT),
    )(x, w_scaled, shift)

    return out
